# baseline (speedup 1.0000x reference)
import functools

import jax
import jax.numpy as jnp
from jax import lax
from jax.experimental import pallas as pl
from jax.experimental.pallas import tpu as pltpu
from jax.experimental.pallas import tpu_sc as plsc

_B, _S, _V, _D, _H, _T = 128, 200, 100000, 64, 64, 4
_ROWS = _B * _S
_NW = 32
_RPW = _ROWS // _NW
_NCH = 8
_CH = _RPW // _NCH

_CHUNK = 8
_NGRID = _S // _CHUNK
_VB = 2048
_NVB = (_V + _VB - 1) // _VB

_PREC = lax.Precision.HIGHEST


def _dot(a, b):
    return jnp.dot(a, b, preferred_element_type=jnp.float32, precision=_PREC)


def _gather_body(idx_hbm, table_hbm, out_hbm, idx_v, rows_v, sem):
    nc = plsc.get_sparse_core_info().num_cores
    wid = lax.axis_index("s") * nc + lax.axis_index("c")
    pltpu.sync_copy(idx_hbm.at[wid], idx_v)
    copies = [
        pltpu.async_copy(table_hbm.at[idx_v.at[j]],
                         rows_v.at[pl.ds(j * _CH, _CH)], sem)
        for j in range(_NCH)
    ]
    for c in copies:
        c.wait()
    pltpu.sync_copy(rows_v, out_hbm.at[pl.ds(wid * _RPW, _RPW)])


@jax.jit
def _sc_gather(idx, table2):
    k = pl.kernel(
        _gather_body,
        mesh=plsc.VectorSubcoreMesh(core_axis_name="c", subcore_axis_name="s"),
        out_type=jax.ShapeDtypeStruct((_ROWS, 2 * _D), jnp.float32),
        scratch_types=[
            pltpu.VMEM((_NCH, _CH), jnp.int32),
            pltpu.VMEM((_RPW, 2 * _D), jnp.float32),
            pltpu.SemaphoreType.DMA,
        ],
    )
    return k(idx.reshape(_NW, _NCH, _CH), table2)


def _scan_body(xe_ref, par_ref, wx_ref, wh_ref, b_ref, wf_ref, bf_ref,
               dts_ref, hs_ref, h_ref):
    i = pl.program_id(0)

    @pl.when(i == 0)
    def _():
        h_ref[...] = jnp.zeros_like(h_ref)

    par = par_ref[0]
    sels = []
    for t in range(_CHUNK):
        xt = xe_ref[t]
        p = par[:, t:t + 1]
        lo = xt[:, :_D]
        hi = xt[:, _D:]
        sels.append(lo + p * (hi - lo))
    xe_c = jnp.concatenate(sels, axis=0)
    gx = _dot(xe_c, wx_ref[...]) + b_ref[...]

    h = h_ref[...]
    for t in range(_CHUNK):
        gx_t = gx[t * _B:(t + 1) * _B]
        gh = _dot(h, wh_ref[...])
        zr = jax.nn.sigmoid(gx_t[:, :2 * _H] + gh[:, :2 * _H])
        z = zr[:, :_H]
        r = zr[:, _H:]
        n = jnp.tanh(gx_t[:, 2 * _H:] + r * gh[:, 2 * _H:])
        h = (1.0 - z) * h + z * n
    h_ref[...] = h

    @pl.when(i == _NGRID - 1)
    def _():
        def f(hh):
            return jnp.tanh(_dot(hh, wf_ref[...]) + bf_ref[...])

        hs_ref[0:_B, :] = h
        hc = h
        for s in range(_T - 1):
            dt = dts_ref[s]
            k1 = f(hc)
            k2 = f(hc + 0.5 * dt * k1)
            k3 = f(hc + 0.5 * dt * k2)
            k4 = f(hc + dt * k3)
            hc = hc + (dt / 6.0) * (k1 + 2.0 * k2 + 2.0 * k3 + k4)
            hs_ref[(s + 1) * _B:(s + 2) * _B, :] = hc


@jax.jit
def _scan_call(xe2, par, Wx, Wh, b, Wf, bf, dts):
    return pl.pallas_call(
        _scan_body,
        grid=(_NGRID,),
        in_specs=[
            pl.BlockSpec((_CHUNK, _B, 2 * _D), lambda i: (i, 0, 0)),
            pl.BlockSpec((1, _B, _CHUNK), lambda i: (i, 0, 0)),
            pl.BlockSpec((_D, 3 * _H), lambda i: (0, 0)),
            pl.BlockSpec((_H, 3 * _H), lambda i: (0, 0)),
            pl.BlockSpec((1, 3 * _H), lambda i: (0, 0)),
            pl.BlockSpec((_H, _H), lambda i: (0, 0)),
            pl.BlockSpec((1, _H), lambda i: (0, 0)),
            pl.BlockSpec(memory_space=pltpu.SMEM),
        ],
        out_specs=pl.BlockSpec((_T * _B, _H), lambda i: (0, 0)),
        out_shape=jax.ShapeDtypeStruct((_T * _B, _H), jnp.float32),
        scratch_shapes=[pltpu.VMEM((_B, _H), jnp.float32)],
    )(xe2, par, Wx, Wh, b, Wf, bf, dts)


def _proj_body(hs_ref, wout_ref, bout_ref, out_ref):
    out_ref[...] = _dot(hs_ref[...], wout_ref[...]) + bout_ref[...]


@jax.jit
def _proj_call(hs, Wout, bout):
    return pl.pallas_call(
        _proj_body,
        grid=(_NVB,),
        in_specs=[
            pl.BlockSpec((_T * _B, _H), lambda j: (0, 0)),
            pl.BlockSpec((_H, _VB), lambda j: (0, j)),
            pl.BlockSpec((1, _VB), lambda j: (0, j)),
        ],
        out_specs=pl.BlockSpec((_T * _B, _VB), lambda j: (0, j)),
        out_shape=jax.ShapeDtypeStruct((_T * _B, _V), jnp.float32),
    )(hs, Wout, bout)


def kernel(x, t_span, emb_table, Wx, Wh, b, Wf, bf, Wout, bout):
    xi = x.astype(jnp.int32)
    idxp = xi.T.reshape(-1) >> 1
    par = jnp.transpose((xi & 1).astype(jnp.float32).reshape(_B, _NGRID, _CHUNK),
                        (1, 0, 2))
    table2 = emb_table.reshape(_V // 2, 2 * _D)
    xe2 = _sc_gather(idxp, table2)
    dts = t_span[1:] - t_span[:-1]
    hs = _scan_call(xe2.reshape(_S, _B, 2 * _D), par, Wx, Wh,
                    b.reshape(1, -1), Wf, bf.reshape(1, -1), dts)
    out = _proj_call(hs, Wout, bout.reshape(1, -1))
    return out.reshape(_T, _B, _V)


# speedup vs baseline: 1.4056x; 1.4056x over previous
import functools

import jax
import jax.numpy as jnp
from jax import lax
from jax.experimental import pallas as pl
from jax.experimental.pallas import tpu as pltpu
from jax.experimental.pallas import tpu_sc as plsc

_B, _S, _V, _D, _H, _T = 128, 200, 100000, 64, 64, 4
_ROWS = _B * _S
_NW = 32
_RPW = _ROWS // _NW
_NCH = 8
_CH = _RPW // _NCH

_CHUNK = 8
_NGRID = _S // _CHUNK
_VB = 2048
_NVB = (_V + _VB - 1) // _VB

def _dot(a, b):
    return jnp.dot(a, b, preferred_element_type=jnp.float32)


def _sigmoid(u):
    return 0.5 * jnp.tanh(0.5 * u) + 0.5


_RB = 2000
_NRB = _V // (2 * _RB)


def _repack_body(lo_ref, hi_ref, t2_ref):
    t2_ref[:, :_D] = lo_ref[...]
    t2_ref[:, _D:] = hi_ref[...]


@jax.jit
def _repack(emb):
    return pl.pallas_call(
        _repack_body,
        grid=(_NRB,),
        in_specs=[
            pl.BlockSpec((_RB, _D), lambda i: (i, 0)),
            pl.BlockSpec((_RB, _D), lambda i: (i + _NRB, 0)),
        ],
        out_specs=pl.BlockSpec((_RB, 2 * _D), lambda i: (i, 0)),
        out_shape=jax.ShapeDtypeStruct((_V // 2, 2 * _D), jnp.float32),
    )(emb, emb)


def _gather_body(idx_hbm, table_hbm, out_hbm, idx_v, rows_v, sem):
    nc = plsc.get_sparse_core_info().num_cores
    wid = lax.axis_index("s") * nc + lax.axis_index("c")
    pltpu.sync_copy(idx_hbm.at[wid], idx_v)
    copies = [
        pltpu.async_copy(table_hbm.at[idx_v.at[j]],
                         rows_v.at[pl.ds(j * _CH, _CH)], sem)
        for j in range(_NCH)
    ]
    for c in copies:
        c.wait()
    pltpu.sync_copy(rows_v, out_hbm.at[pl.ds(wid * _RPW, _RPW)])


@jax.jit
def _sc_gather(idx, table2):
    k = pl.kernel(
        _gather_body,
        mesh=plsc.VectorSubcoreMesh(core_axis_name="c", subcore_axis_name="s"),
        out_type=jax.ShapeDtypeStruct((_ROWS, 2 * _D), jnp.float32),
        scratch_types=[
            pltpu.VMEM((_NCH, _CH), jnp.int32),
            pltpu.VMEM((_RPW, 2 * _D), jnp.float32),
            pltpu.SemaphoreType.DMA,
        ],
    )
    return k(idx.reshape(_NW, _NCH, _CH), table2)


def _scan_body(xe_ref, par_ref, wx_ref, wh_ref, b_ref, wf_ref, bf_ref,
               dts_ref, hs_ref, h_ref):
    i = pl.program_id(0)

    @pl.when(i == 0)
    def _():
        h_ref[...] = jnp.zeros_like(h_ref)

    par = par_ref[0]
    sels = []
    for t in range(_CHUNK):
        xt = xe_ref[t]
        p = par[:, t:t + 1]
        lo = xt[:, :_D]
        hi = xt[:, _D:]
        sels.append(lo + p * (hi - lo))
    xe_c = jnp.concatenate(sels, axis=0)
    gx = _dot(xe_c, wx_ref[...]) + b_ref[...]

    h = h_ref[...]
    for t in range(_CHUNK):
        gx_t = gx[t * _B:(t + 1) * _B]
        gh = _dot(h, wh_ref[...])
        zr = _sigmoid(gx_t[:, :2 * _H] + gh[:, :2 * _H])
        z = zr[:, :_H]
        r = zr[:, _H:]
        n = jnp.tanh(gx_t[:, 2 * _H:] + r * gh[:, 2 * _H:])
        h = (1.0 - z) * h + z * n
    h_ref[...] = h

    @pl.when(i == _NGRID - 1)
    def _():
        def f(hh):
            return jnp.tanh(_dot(hh, wf_ref[...]) + bf_ref[...])

        hs_ref[0:_B, :] = h
        hc = h
        for s in range(_T - 1):
            dt = dts_ref[s]
            k1 = f(hc)
            k2 = f(hc + 0.5 * dt * k1)
            k3 = f(hc + 0.5 * dt * k2)
            k4 = f(hc + dt * k3)
            hc = hc + (dt / 6.0) * (k1 + 2.0 * k2 + 2.0 * k3 + k4)
            hs_ref[(s + 1) * _B:(s + 2) * _B, :] = hc


@jax.jit
def _scan_call(xe2, par, Wx, Wh, b, Wf, bf, dts):
    return pl.pallas_call(
        _scan_body,
        grid=(_NGRID,),
        in_specs=[
            pl.BlockSpec((_CHUNK, _B, 2 * _D), lambda i: (i, 0, 0)),
            pl.BlockSpec((1, _B, _CHUNK), lambda i: (i, 0, 0)),
            pl.BlockSpec((_D, 3 * _H), lambda i: (0, 0)),
            pl.BlockSpec((_H, 3 * _H), lambda i: (0, 0)),
            pl.BlockSpec((1, 3 * _H), lambda i: (0, 0)),
            pl.BlockSpec((_H, _H), lambda i: (0, 0)),
            pl.BlockSpec((1, _H), lambda i: (0, 0)),
            pl.BlockSpec(memory_space=pltpu.SMEM),
        ],
        out_specs=pl.BlockSpec((_T * _B, _H), lambda i: (0, 0)),
        out_shape=jax.ShapeDtypeStruct((_T * _B, _H), jnp.float32),
        scratch_shapes=[pltpu.VMEM((_B, _H), jnp.float32)],
    )(xe2, par, Wx, Wh, b, Wf, bf, dts)


def _proj_body(hs_ref, wout_ref, bout_ref, out_ref):
    out_ref[...] = _dot(hs_ref[...], wout_ref[...]) + bout_ref[...]


@jax.jit
def _proj_call(hs, Wout, bout):
    return pl.pallas_call(
        _proj_body,
        grid=(_NVB,),
        in_specs=[
            pl.BlockSpec((_T * _B, _H), lambda j: (0, 0)),
            pl.BlockSpec((_H, _VB), lambda j: (0, j)),
            pl.BlockSpec((1, _VB), lambda j: (0, j)),
        ],
        out_specs=pl.BlockSpec((_T * _B, _VB), lambda j: (0, j)),
        out_shape=jax.ShapeDtypeStruct((_T * _B, _V), jnp.float32),
    )(hs, Wout, bout)


def kernel(x, t_span, emb_table, Wx, Wh, b, Wf, bf, Wout, bout):
    xi = x.astype(jnp.int32)
    xt = xi.T.reshape(-1)
    idxp = jnp.where(xt < _V // 2, xt, xt - _V // 2)
    par = jnp.transpose(
        (xi >= _V // 2).astype(jnp.float32).reshape(_B, _NGRID, _CHUNK),
        (1, 0, 2))
    table2 = _repack(emb_table)
    xe2 = _sc_gather(idxp, table2)
    dts = t_span[1:] - t_span[:-1]
    hs = _scan_call(xe2.reshape(_S, _B, 2 * _D), par, Wx, Wh,
                    b.reshape(1, -1), Wf, bf.reshape(1, -1), dts)
    out = _proj_call(hs, Wout, bout.reshape(1, -1))
    return out.reshape(_T, _B, _V)


# speedup vs baseline: 2.1929x; 1.5601x over previous
import functools

import jax
import jax.numpy as jnp
from jax import lax
from jax.experimental import pallas as pl
from jax.experimental.pallas import tpu as pltpu
from jax.experimental.pallas import tpu_sc as plsc

_B, _S, _V, _D, _H, _T = 128, 200, 100000, 64, 64, 4
_ROWS = _B * _S
_NW = 32
_RPW = _ROWS // _NW
_NCH = 8
_CH = _RPW // _NCH

_CHUNK = 8
_NGRID = _S // _CHUNK
_VB = 2048
_NVB = (_V + _VB - 1) // _VB

def _dot(a, b):
    return jnp.dot(a, b, preferred_element_type=jnp.float32)


def _sigmoid(u):
    return 0.5 * jnp.tanh(0.5 * u) + 0.5


_RB = 2000
_NRB = _V // (2 * _RB)


def _repack_body(lo_ref, hi_ref, t2_ref):
    t2_ref[:, :_D] = lo_ref[...]
    t2_ref[:, _D:] = hi_ref[...]


@jax.jit
def _repack(emb):
    return pl.pallas_call(
        _repack_body,
        grid=(_NRB,),
        in_specs=[
            pl.BlockSpec((_RB, _D), lambda i: (i, 0)),
            pl.BlockSpec((_RB, _D), lambda i: (i + _NRB, 0)),
        ],
        out_specs=pl.BlockSpec((_RB, 2 * _D), lambda i: (i, 0)),
        out_shape=jax.ShapeDtypeStruct((_V // 2, 2 * _D), jnp.float32),
    )(emb, emb)


def _gather_body(idx_hbm, table_hbm, out_hbm, idx_v, rows_v, sem):
    nc = plsc.get_sparse_core_info().num_cores
    wid = lax.axis_index("s") * nc + lax.axis_index("c")
    pltpu.sync_copy(idx_hbm.at[wid], idx_v)
    copies = [
        pltpu.async_copy(table_hbm.at[idx_v.at[j]],
                         rows_v.at[pl.ds(j * _CH, _CH)], sem)
        for j in range(_NCH)
    ]
    for c in copies:
        c.wait()
    pltpu.sync_copy(rows_v, out_hbm.at[pl.ds(wid * _RPW, _RPW)])


@jax.jit
def _sc_gather(idx, table2):
    k = pl.kernel(
        _gather_body,
        mesh=plsc.VectorSubcoreMesh(core_axis_name="c", subcore_axis_name="s"),
        out_type=jax.ShapeDtypeStruct((_ROWS, 2 * _D), jnp.float32),
        scratch_types=[
            pltpu.VMEM((_NCH, _CH), jnp.int32),
            pltpu.VMEM((_RPW, 2 * _D), jnp.float32),
            pltpu.SemaphoreType.DMA,
        ],
    )
    return k(idx.reshape(_NW, _NCH, _CH), table2)


def _scan_body(xe_ref, par_ref, wx_ref, wh_ref, b_ref, wf_ref, bf_ref,
               dts_ref, hs_ref, h_ref):
    i = pl.program_id(0)

    @pl.when(i == 0)
    def _():
        h_ref[...] = jnp.zeros_like(h_ref)

    par = par_ref[0]
    sels = []
    for t in range(_CHUNK):
        xt = xe_ref[t]
        p = par[:, t:t + 1]
        lo = xt[:, :_D]
        hi = xt[:, _D:]
        sels.append(lo + p * (hi - lo))
    xe_c = jnp.concatenate(sels, axis=0)
    gx = _dot(xe_c, wx_ref[...]) + b_ref[...]

    h = h_ref[...]
    for t in range(_CHUNK):
        gx_t = gx[t * _B:(t + 1) * _B]
        gh = _dot(h, wh_ref[...])
        zr = _sigmoid(gx_t[:, :2 * _H] + gh[:, :2 * _H])
        z = zr[:, :_H]
        r = zr[:, _H:]
        n = jnp.tanh(gx_t[:, 2 * _H:] + r * gh[:, 2 * _H:])
        h = (1.0 - z) * h + z * n
    h_ref[...] = h

    @pl.when(i == _NGRID - 1)
    def _():
        def f(hh):
            return jnp.tanh(_dot(hh, wf_ref[...]) + bf_ref[...])

        hs_ref[0:_B, :] = h
        hc = h
        for s in range(_T - 1):
            dt = dts_ref[s]
            k1 = f(hc)
            k2 = f(hc + 0.5 * dt * k1)
            k3 = f(hc + 0.5 * dt * k2)
            k4 = f(hc + dt * k3)
            hc = hc + (dt / 6.0) * (k1 + 2.0 * k2 + 2.0 * k3 + k4)
            hs_ref[(s + 1) * _B:(s + 2) * _B, :] = hc


@jax.jit
def _scan_call(xe2, par, Wx, Wh, b, Wf, bf, dts):
    return pl.pallas_call(
        _scan_body,
        grid=(_NGRID,),
        in_specs=[
            pl.BlockSpec((_CHUNK, _B, 2 * _D), lambda i: (i, 0, 0)),
            pl.BlockSpec((1, _B, _CHUNK), lambda i: (i, 0, 0)),
            pl.BlockSpec((_D, 3 * _H), lambda i: (0, 0)),
            pl.BlockSpec((_H, 3 * _H), lambda i: (0, 0)),
            pl.BlockSpec((1, 3 * _H), lambda i: (0, 0)),
            pl.BlockSpec((_H, _H), lambda i: (0, 0)),
            pl.BlockSpec((1, _H), lambda i: (0, 0)),
            pl.BlockSpec(memory_space=pltpu.SMEM),
        ],
        out_specs=pl.BlockSpec((_T * _B, _H), lambda i: (0, 0)),
        out_shape=jax.ShapeDtypeStruct((_T * _B, _H), jnp.float32),
        scratch_shapes=[pltpu.VMEM((_B, _H), jnp.float32)],
    )(xe2, par, Wx, Wh, b, Wf, bf, dts)


def _proj_body(hs_ref, wout_ref, bout_ref, out_ref):
    w = wout_ref[...]
    ones = jnp.ones((_B, 1), jnp.float32)
    bias = jax.lax.dot_general(
        bout_ref[...], ones, (((0,), (1,)), ((), ())),
        preferred_element_type=jnp.float32)
    for t in range(_T):
        hs_t = hs_ref[t * _B:(t + 1) * _B]
        out_ref[t] = jax.lax.dot_general(
            w, hs_t, (((0,), (1,)), ((), ())),
            preferred_element_type=jnp.float32) + bias


@jax.jit
def _proj_call(hs, Wout, bout):
    return pl.pallas_call(
        _proj_body,
        grid=(_NVB,),
        in_specs=[
            pl.BlockSpec((_T * _B, _H), lambda j: (0, 0)),
            pl.BlockSpec((_H, _VB), lambda j: (0, j)),
            pl.BlockSpec((1, _VB), lambda j: (0, j)),
        ],
        out_specs=pl.BlockSpec((_T, _VB, _B), lambda j: (0, j, 0)),
        out_shape=jax.ShapeDtypeStruct((_T, _V, _B), jnp.float32),
    )(hs, Wout, bout)


def kernel(x, t_span, emb_table, Wx, Wh, b, Wf, bf, Wout, bout):
    xi = x.astype(jnp.int32)
    xt = xi.T.reshape(-1)
    idxp = jnp.where(xt < _V // 2, xt, xt - _V // 2)
    par = jnp.transpose(
        (xi >= _V // 2).astype(jnp.float32).reshape(_B, _NGRID, _CHUNK),
        (1, 0, 2))
    table2 = _repack(emb_table)
    xe2 = _sc_gather(idxp, table2)
    dts = t_span[1:] - t_span[:-1]
    hs = _scan_call(xe2.reshape(_S, _B, 2 * _D), par, Wx, Wh,
                    b.reshape(1, -1), Wf, bf.reshape(1, -1), dts)
    out = _proj_call(hs, Wout, bout.reshape(1, -1))
    return jnp.transpose(out, (0, 2, 1))


# speedup vs baseline: 2.2583x; 1.0298x over previous
import functools

import jax
import jax.numpy as jnp
from jax import lax
from jax.experimental import pallas as pl
from jax.experimental.pallas import tpu as pltpu
from jax.experimental.pallas import tpu_sc as plsc

_B, _S, _V, _D, _H, _T = 128, 200, 100000, 64, 64, 4
_ROWS = _B * _S
_NW = 32
_RPW = _ROWS // _NW
_NCH = 8
_CH = _RPW // _NCH

_CHUNK = 8
_NGRID = _S // _CHUNK
_NSPL = 4
_VB = 2048
_NVB = (_V + _VB - 1) // _VB

def _dot(a, b):
    return jnp.dot(a, b, preferred_element_type=jnp.float32)


def _sigmoid(u):
    return 0.5 * jnp.tanh(0.5 * u) + 0.5


_RB = 2000
_NRB = _V // (2 * _RB)


def _repack_body(lo_ref, hi_ref, t2_ref):
    t2_ref[:, :_D] = lo_ref[...]
    t2_ref[:, _D:] = hi_ref[...]


@jax.jit
def _repack(emb):
    return pl.pallas_call(
        _repack_body,
        grid=(_NRB,),
        in_specs=[
            pl.BlockSpec((_RB, _D), lambda i: (i, 0)),
            pl.BlockSpec((_RB, _D), lambda i: (i + _NRB, 0)),
        ],
        out_specs=pl.BlockSpec((_RB, 2 * _D), lambda i: (i, 0)),
        out_shape=jax.ShapeDtypeStruct((_V // 2, 2 * _D), jnp.float32),
    )(emb, emb)


def _gather_body(idx_hbm, table_hbm, out_hbm, idx_v, rows_v, sem):
    nc = plsc.get_sparse_core_info().num_cores
    wid = lax.axis_index("s") * nc + lax.axis_index("c")
    pltpu.sync_copy(idx_hbm.at[wid], idx_v)
    copies = [
        pltpu.async_copy(table_hbm.at[idx_v.at[j]],
                         rows_v.at[pl.ds(j * _CH, _CH)], sem)
        for j in range(_NCH)
    ]
    for c in copies:
        c.wait()
    pltpu.sync_copy(rows_v, out_hbm.at[pl.ds(wid * _RPW, _RPW)])


@jax.jit
def _sc_gather(idx, table2):
    k = pl.kernel(
        _gather_body,
        mesh=plsc.VectorSubcoreMesh(core_axis_name="c", subcore_axis_name="s"),
        out_type=jax.ShapeDtypeStruct((_ROWS, 2 * _D), jnp.float32),
        scratch_types=[
            pltpu.VMEM((_NCH, _CH), jnp.int32),
            pltpu.VMEM((_RPW, 2 * _D), jnp.float32),
            pltpu.SemaphoreType.DMA,
        ],
    )
    return k(idx.reshape(_NW, _NCH, _CH), table2)


def _scan_body(xe_ref, par_ref, wxz_ref, wxr_ref, wxn_ref,
               whz_ref, whr_ref, whn_ref, bz_ref, br_ref, bn_ref,
               wf_ref, bf_ref, dts_ref, hs_ref, h_ref):
    i = pl.program_id(0)

    @pl.when(i == 0)
    def _():
        h_ref[...] = jnp.zeros_like(h_ref)

    par = par_ref[0]
    sels = []
    for t in range(_CHUNK):
        xt = xe_ref[t]
        p = par[:, t:t + 1]
        lo = xt[:, :_D]
        hi = xt[:, _D:]
        sels.append(lo + p * (hi - lo))
    xe_c = jnp.concatenate(sels, axis=0)
    gxz = _dot(xe_c, wxz_ref[...]) + bz_ref[...]
    gxr = _dot(xe_c, wxr_ref[...]) + br_ref[...]
    gxn = _dot(xe_c, wxn_ref[...]) + bn_ref[...]

    whz, whr, whn = whz_ref[...], whr_ref[...], whn_ref[...]
    h = h_ref[...]
    for t in range(_CHUNK):
        lo_ = t * _B
        hi_ = (t + 1) * _B
        z = _sigmoid(gxz[lo_:hi_] + _dot(h, whz))
        r = _sigmoid(gxr[lo_:hi_] + _dot(h, whr))
        n = jnp.tanh(gxn[lo_:hi_] + r * _dot(h, whn))
        h = h + z * (n - h)
    h_ref[...] = h

    @pl.when(i == _NGRID - 1)
    def _():
        def f(hh):
            return jnp.tanh(_dot(hh, wf_ref[...]) + bf_ref[...])

        hs_ref[0:_B, :] = h
        hc = h
        for s in range(_T - 1):
            dt = dts_ref[s]
            k1 = f(hc)
            k2 = f(hc + 0.5 * dt * k1)
            k3 = f(hc + 0.5 * dt * k2)
            k4 = f(hc + dt * k3)
            hc = hc + (dt / 6.0) * (k1 + 2.0 * k2 + 2.0 * k3 + k4)
            hs_ref[(s + 1) * _B:(s + 2) * _B, :] = hc


def _wspec():
    return pl.BlockSpec((_H, _H), lambda i: (0, 0))


def _bspec():
    return pl.BlockSpec((1, _H), lambda i: (0, 0))


@jax.jit
def _scan_call(xe2, par, Wx, Wh, b, Wf, bf, dts):
    b2 = b.reshape(1, 3 * _H)
    parts = []
    for g in range(3):
        parts += [Wx[:, g * _H:(g + 1) * _H], Wh[:, g * _H:(g + 1) * _H],
                  b2[:, g * _H:(g + 1) * _H]]
    wxz, whz, bz, wxr, whr, br, wxn, whn, bn = parts
    return pl.pallas_call(
        _scan_body,
        grid=(_NGRID,),
        in_specs=[
            pl.BlockSpec((_CHUNK, _B, 2 * _D), lambda i: (i, 0, 0)),
            pl.BlockSpec((1, _B, _CHUNK), lambda i: (i, 0, 0)),
            _wspec(), _wspec(), _wspec(),
            _wspec(), _wspec(), _wspec(),
            _bspec(), _bspec(), _bspec(),
            _wspec(), _bspec(),
            pl.BlockSpec(memory_space=pltpu.SMEM),
        ],
        out_specs=pl.BlockSpec((_T * _B, _H), lambda i: (0, 0)),
        out_shape=jax.ShapeDtypeStruct((_T * _B, _H), jnp.float32),
        scratch_shapes=[pltpu.VMEM((_B, _H), jnp.float32)],
    )(xe2, par, wxz, wxr, wxn, whz, whr, whn, bz, br, bn, Wf, bf, dts)


def _proj_body(hs_ref, wout_ref, bout_ref, out_ref):
    w = wout_ref[...]
    ones = jnp.ones((_B, 1), jnp.float32)
    bias = jax.lax.dot_general(
        bout_ref[...], ones, (((0,), (1,)), ((), ())),
        preferred_element_type=jnp.float32)
    for t in range(_T):
        hs_t = hs_ref[t * _B:(t + 1) * _B]
        out_ref[t] = jax.lax.dot_general(
            w, hs_t, (((0,), (1,)), ((), ())),
            preferred_element_type=jnp.float32) + bias


@jax.jit
def _proj_call(hs, Wout, bout):
    return pl.pallas_call(
        _proj_body,
        grid=(_NVB,),
        in_specs=[
            pl.BlockSpec((_T * _B, _H), lambda j: (0, 0)),
            pl.BlockSpec((_H, _VB), lambda j: (0, j)),
            pl.BlockSpec((1, _VB), lambda j: (0, j)),
        ],
        out_specs=pl.BlockSpec((_T, _VB, _B), lambda j: (0, j, 0)),
        out_shape=jax.ShapeDtypeStruct((_T, _V, _B), jnp.float32),
    )(hs, Wout, bout)


def kernel(x, t_span, emb_table, Wx, Wh, b, Wf, bf, Wout, bout):
    xi = x.astype(jnp.int32)
    xt = xi.T.reshape(-1)
    idxp = jnp.where(xt < _V // 2, xt, xt - _V // 2)
    par = jnp.transpose(
        (xi >= _V // 2).astype(jnp.float32).reshape(_B, _NGRID, _CHUNK),
        (1, 0, 2))
    table2 = _repack(emb_table)
    xe2 = _sc_gather(idxp, table2)
    dts = t_span[1:] - t_span[:-1]
    hs = _scan_call(xe2.reshape(_S, _B, 2 * _D), par, Wx, Wh,
                    b.reshape(1, -1), Wf, bf.reshape(1, -1), dts)
    out = _proj_call(hs, Wout, bout.reshape(1, -1))
    return jnp.transpose(out, (0, 2, 1))


# speedup vs baseline: 2.2713x; 1.0058x over previous
import functools

import jax
import jax.numpy as jnp
from jax import lax
from jax.experimental import pallas as pl
from jax.experimental.pallas import tpu as pltpu
from jax.experimental.pallas import tpu_sc as plsc

_B, _S, _V, _D, _H, _T = 128, 200, 100000, 64, 64, 4
_ROWS = _B * _S
_NW = 32
_RPW = _ROWS // _NW
_NCH = 8
_CH = _RPW // _NCH

_CHUNK = 100
_NGRID = _S // _CHUNK
_NSPL = 4
_VB = 2048
_NVB = (_V + _VB - 1) // _VB

def _dot(a, b):
    return jnp.dot(a, b, preferred_element_type=jnp.float32)


def _sigmoid(u):
    return 0.5 * jnp.tanh(0.5 * u) + 0.5


_RB = 2000
_NRB = _V // (2 * _RB)


def _repack_body(lo_ref, hi_ref, t2_ref):
    t2_ref[:, :_D] = lo_ref[...]
    t2_ref[:, _D:] = hi_ref[...]


@jax.jit
def _repack(emb):
    return pl.pallas_call(
        _repack_body,
        grid=(_NRB,),
        in_specs=[
            pl.BlockSpec((_RB, _D), lambda i: (i, 0)),
            pl.BlockSpec((_RB, _D), lambda i: (i + _NRB, 0)),
        ],
        out_specs=pl.BlockSpec((_RB, 2 * _D), lambda i: (i, 0)),
        out_shape=jax.ShapeDtypeStruct((_V // 2, 2 * _D), jnp.float32),
    )(emb, emb)


def _gather_body(idx_hbm, table_hbm, out_hbm, idx_v, rows_v, sem):
    nc = plsc.get_sparse_core_info().num_cores
    wid = lax.axis_index("s") * nc + lax.axis_index("c")
    pltpu.sync_copy(idx_hbm.at[wid], idx_v)
    copies = [
        pltpu.async_copy(table_hbm.at[idx_v.at[j]],
                         rows_v.at[pl.ds(j * _CH, _CH)], sem)
        for j in range(_NCH)
    ]
    for c in copies:
        c.wait()
    pltpu.sync_copy(rows_v, out_hbm.at[pl.ds(wid * _RPW, _RPW)])


@jax.jit
def _sc_gather(idx, table2):
    k = pl.kernel(
        _gather_body,
        mesh=plsc.VectorSubcoreMesh(core_axis_name="c", subcore_axis_name="s"),
        out_type=jax.ShapeDtypeStruct((_ROWS, 2 * _D), jnp.float32),
        scratch_types=[
            pltpu.VMEM((_NCH, _CH), jnp.int32),
            pltpu.VMEM((_RPW, 2 * _D), jnp.float32),
            pltpu.SemaphoreType.DMA,
        ],
    )
    return k(idx.reshape(_NW, _NCH, _CH), table2)


def _scan_body(xe_ref, par_ref, wxz_ref, wxr_ref, wxn_ref,
               whz_ref, whr_ref, whn_ref, bz_ref, br_ref, bn_ref,
               wf_ref, bf_ref, dts_ref, hs_ref, h_ref):
    i = pl.program_id(0)

    @pl.when(i == 0)
    def _():
        h_ref[...] = jnp.zeros_like(h_ref)

    par = par_ref[0]
    sels = []
    for t in range(_CHUNK):
        xt = xe_ref[t]
        p = par[:, t:t + 1]
        lo = xt[:, :_D]
        hi = xt[:, _D:]
        sels.append(lo + p * (hi - lo))
    xe_c = jnp.concatenate(sels, axis=0)
    gxz = _dot(xe_c, wxz_ref[...]) + bz_ref[...]
    gxr = _dot(xe_c, wxr_ref[...]) + br_ref[...]
    gxn = _dot(xe_c, wxn_ref[...]) + bn_ref[...]

    whz, whr, whn = whz_ref[...], whr_ref[...], whn_ref[...]

    def dots(hh):
        return _dot(hh, whz), _dot(hh, whr), _dot(hh, whn)

    def gates(t, base, gh, hh):
        lo_ = t * _B + base
        hi_ = lo_ + _B // 2
        z = _sigmoid(gxz[lo_:hi_] + gh[0])
        r = _sigmoid(gxr[lo_:hi_] + gh[1])
        n = jnp.tanh(gxn[lo_:hi_] + r * gh[2])
        return hh + z * (n - hh)

    ha = h_ref[0:_B // 2, :]
    hb = h_ref[_B // 2:_B, :]
    gha = dots(ha)
    for t in range(_CHUNK):
        ghb = dots(hb)
        ha = gates(t, 0, gha, ha)
        if t + 1 < _CHUNK:
            gha = dots(ha)
        hb = gates(t, _B // 2, ghb, hb)
    h_ref[0:_B // 2, :] = ha
    h_ref[_B // 2:_B, :] = hb

    @pl.when(i == _NGRID - 1)
    def _():
        def f(hh):
            return jnp.tanh(_dot(hh, wf_ref[...]) + bf_ref[...])

        h = jnp.concatenate([ha, hb], axis=0)
        hs_ref[0:_B, :] = h
        hc = h
        for s in range(_T - 1):
            dt = dts_ref[s]
            k1 = f(hc)
            k2 = f(hc + 0.5 * dt * k1)
            k3 = f(hc + 0.5 * dt * k2)
            k4 = f(hc + dt * k3)
            hc = hc + (dt / 6.0) * (k1 + 2.0 * k2 + 2.0 * k3 + k4)
            hs_ref[(s + 1) * _B:(s + 2) * _B, :] = hc


def _wspec():
    return pl.BlockSpec((_H, _H), lambda i: (0, 0))


def _bspec():
    return pl.BlockSpec((1, _H), lambda i: (0, 0))


@jax.jit
def _scan_call(xe2, par, Wx, Wh, b, Wf, bf, dts):
    b2 = b.reshape(1, 3 * _H)
    parts = []
    for g in range(3):
        parts += [Wx[:, g * _H:(g + 1) * _H], Wh[:, g * _H:(g + 1) * _H],
                  b2[:, g * _H:(g + 1) * _H]]
    wxz, whz, bz, wxr, whr, br, wxn, whn, bn = parts
    return pl.pallas_call(
        _scan_body,
        grid=(_NGRID,),
        in_specs=[
            pl.BlockSpec((_CHUNK, _B, 2 * _D), lambda i: (i, 0, 0)),
            pl.BlockSpec((1, _B, _CHUNK), lambda i: (i, 0, 0)),
            _wspec(), _wspec(), _wspec(),
            _wspec(), _wspec(), _wspec(),
            _bspec(), _bspec(), _bspec(),
            _wspec(), _bspec(),
            pl.BlockSpec(memory_space=pltpu.SMEM),
        ],
        out_specs=pl.BlockSpec((_T * _B, _H), lambda i: (0, 0)),
        out_shape=jax.ShapeDtypeStruct((_T * _B, _H), jnp.float32),
        scratch_shapes=[pltpu.VMEM((_B, _H), jnp.float32)],
    )(xe2, par, wxz, wxr, wxn, whz, whr, whn, bz, br, bn, Wf, bf, dts)


def _proj_body(hs_ref, wout_ref, bout_ref, out_ref):
    w = wout_ref[...]
    ones = jnp.ones((_B, 1), jnp.float32)
    bias = jax.lax.dot_general(
        bout_ref[...], ones, (((0,), (1,)), ((), ())),
        preferred_element_type=jnp.float32)
    for t in range(_T):
        hs_t = hs_ref[t * _B:(t + 1) * _B]
        out_ref[t] = jax.lax.dot_general(
            w, hs_t, (((0,), (1,)), ((), ())),
            preferred_element_type=jnp.float32) + bias


@jax.jit
def _proj_call(hs, Wout, bout):
    return pl.pallas_call(
        _proj_body,
        grid=(_NVB,),
        in_specs=[
            pl.BlockSpec((_T * _B, _H), lambda j: (0, 0)),
            pl.BlockSpec((_H, _VB), lambda j: (0, j)),
            pl.BlockSpec((1, _VB), lambda j: (0, j)),
        ],
        out_specs=pl.BlockSpec((_T, _VB, _B), lambda j: (0, j, 0)),
        out_shape=jax.ShapeDtypeStruct((_T, _V, _B), jnp.float32),
    )(hs, Wout, bout)


def kernel(x, t_span, emb_table, Wx, Wh, b, Wf, bf, Wout, bout):
    xi = x.astype(jnp.int32)
    xt = xi.T.reshape(-1)
    idxp = jnp.where(xt < _V // 2, xt, xt - _V // 2)
    par = jnp.transpose(
        (xi >= _V // 2).astype(jnp.float32).reshape(_B, _NGRID, _CHUNK),
        (1, 0, 2))
    table2 = _repack(emb_table)
    xe2 = _sc_gather(idxp, table2)
    dts = t_span[1:] - t_span[:-1]
    hs = _scan_call(xe2.reshape(_S, _B, 2 * _D), par, Wx, Wh,
                    b.reshape(1, -1), Wf, bf.reshape(1, -1), dts)
    out = _proj_call(hs, Wout, bout.reshape(1, -1))
    return jnp.transpose(out, (0, 2, 1))


# speedup vs baseline: 2.2924x; 1.0093x over previous
import functools

import jax
import jax.numpy as jnp
from jax import lax
from jax.experimental import pallas as pl
from jax.experimental.pallas import tpu as pltpu
from jax.experimental.pallas import tpu_sc as plsc

_B, _S, _V, _D, _H, _T = 128, 200, 100000, 64, 64, 4
_ROWS = _B * _S
_NW = 32
_RPW = _ROWS // _NW
_NCH = 8
_CH = _RPW // _NCH

_CHUNK = 100
_NGRID = _S // _CHUNK
_NSPL = 4
_VB = 2048
_NVB = (_V + _VB - 1) // _VB

def _dot(a, b):
    return jnp.dot(a, b, preferred_element_type=jnp.float32)


def _sigmoid(u):
    return 0.5 * jnp.tanh(0.5 * u) + 0.5


_RB = 2000
_NRB = _V // (2 * _RB)


def _repack_body(lo_ref, hi_ref, t2_ref):
    t2_ref[:, :_D] = lo_ref[...]
    t2_ref[:, _D:] = hi_ref[...]


@jax.jit
def _repack(emb):
    return pl.pallas_call(
        _repack_body,
        grid=(_NRB,),
        in_specs=[
            pl.BlockSpec((_RB, _D), lambda i: (i, 0)),
            pl.BlockSpec((_RB, _D), lambda i: (i + _NRB, 0)),
        ],
        out_specs=pl.BlockSpec((_RB, 2 * _D), lambda i: (i, 0)),
        out_shape=jax.ShapeDtypeStruct((_V // 2, 2 * _D), jnp.float32),
    )(emb, emb)


def _gather_body(idx_hbm, table_hbm, out_hbm, idx_v, rows_v, sem):
    nc = plsc.get_sparse_core_info().num_cores
    wid = lax.axis_index("s") * nc + lax.axis_index("c")
    pltpu.sync_copy(idx_hbm.at[wid], idx_v)
    copies = [
        pltpu.async_copy(table_hbm.at[idx_v.at[j]],
                         rows_v.at[pl.ds(j * _CH, _CH)], sem)
        for j in range(_NCH)
    ]
    for c in copies:
        c.wait()
    pltpu.sync_copy(rows_v, out_hbm.at[pl.ds(wid * _RPW, _RPW)])


@jax.jit
def _sc_gather(idx, table2):
    k = pl.kernel(
        _gather_body,
        mesh=plsc.VectorSubcoreMesh(core_axis_name="c", subcore_axis_name="s"),
        out_type=jax.ShapeDtypeStruct((_ROWS, 2 * _D), jnp.float32),
        scratch_types=[
            pltpu.VMEM((_NCH, _CH), jnp.int32),
            pltpu.VMEM((_RPW, 2 * _D), jnp.float32),
            pltpu.SemaphoreType.DMA,
        ],
    )
    return k(idx.reshape(_NW, _NCH, _CH), table2)


def _fused_body(xe_ref, par_ref, wxz_ref, wxr_ref, wxn_ref,
                whz_ref, whr_ref, whn_ref, bz_ref, br_ref, bn_ref,
                wf_ref, bf_ref, dts_ref, wout_ref, bout_ref,
                out_ref, h_ref, hs_ref):
    i = pl.program_id(0)

    @pl.when(i == 0)
    def _():
        h_ref[...] = jnp.zeros_like(h_ref)

    @pl.when(i < _NGRID)
    def _scan_phase():
        _scan_chunk(xe_ref, par_ref, wxz_ref, wxr_ref, wxn_ref,
                    whz_ref, whr_ref, whn_ref, bz_ref, br_ref, bn_ref,
                    wf_ref, bf_ref, dts_ref, hs_ref, h_ref, i)

    @pl.when(i >= _NGRID)
    def _proj_phase():
        w = wout_ref[...]
        ones = jnp.ones((_B, 1), jnp.float32)
        bias = jax.lax.dot_general(
            bout_ref[...], ones, (((0,), (1,)), ((), ())),
            preferred_element_type=jnp.float32)
        for t in range(_T):
            hs_t = hs_ref[t * _B:(t + 1) * _B]
            out_ref[t] = jax.lax.dot_general(
                w, hs_t, (((0,), (1,)), ((), ())),
                preferred_element_type=jnp.float32) + bias


def _scan_chunk(xe_ref, par_ref, wxz_ref, wxr_ref, wxn_ref,
                whz_ref, whr_ref, whn_ref, bz_ref, br_ref, bn_ref,
                wf_ref, bf_ref, dts_ref, hs_ref, h_ref, i):

    par = par_ref[0]
    sels = []
    for t in range(_CHUNK):
        xt = xe_ref[t]
        p = par[:, t:t + 1]
        lo = xt[:, :_D]
        hi = xt[:, _D:]
        sels.append(lo + p * (hi - lo))
    xe_c = jnp.concatenate(sels, axis=0)
    gxz = _dot(xe_c, wxz_ref[...]) + bz_ref[...]
    gxr = _dot(xe_c, wxr_ref[...]) + br_ref[...]
    gxn = _dot(xe_c, wxn_ref[...]) + bn_ref[...]

    whz, whr, whn = whz_ref[...], whr_ref[...], whn_ref[...]

    def dots(hh):
        return _dot(hh, whz), _dot(hh, whr), _dot(hh, whn)

    def gates(t, base, gh, hh):
        lo_ = t * _B + base
        hi_ = lo_ + _B // 2
        z = _sigmoid(gxz[lo_:hi_] + gh[0])
        r = _sigmoid(gxr[lo_:hi_] + gh[1])
        n = jnp.tanh(gxn[lo_:hi_] + r * gh[2])
        return hh + z * (n - hh)

    ha = h_ref[0:_B // 2, :]
    hb = h_ref[_B // 2:_B, :]
    gha = dots(ha)
    for t in range(_CHUNK):
        ghb = dots(hb)
        ha = gates(t, 0, gha, ha)
        if t + 1 < _CHUNK:
            gha = dots(ha)
        hb = gates(t, _B // 2, ghb, hb)
    h_ref[0:_B // 2, :] = ha
    h_ref[_B // 2:_B, :] = hb

    @pl.when(i == _NGRID - 1)
    def _():
        def f(hh):
            return jnp.tanh(_dot(hh, wf_ref[...]) + bf_ref[...])

        h = jnp.concatenate([ha, hb], axis=0)
        hs_ref[0:_B, :] = h
        hc = h
        for s in range(_T - 1):
            dt = dts_ref[s]
            k1 = f(hc)
            k2 = f(hc + 0.5 * dt * k1)
            k3 = f(hc + 0.5 * dt * k2)
            k4 = f(hc + dt * k3)
            hc = hc + (dt / 6.0) * (k1 + 2.0 * k2 + 2.0 * k3 + k4)
            hs_ref[(s + 1) * _B:(s + 2) * _B, :] = hc


def _wspec():
    return pl.BlockSpec((_H, _H), lambda i: (0, 0))


def _bspec():
    return pl.BlockSpec((1, _H), lambda i: (0, 0))


@jax.jit
def _fused_call(xe2, par, Wx, Wh, b, Wf, bf, dts, Wout, bout):
    b2 = b.reshape(1, 3 * _H)
    parts = []
    for g in range(3):
        parts += [Wx[:, g * _H:(g + 1) * _H], Wh[:, g * _H:(g + 1) * _H],
                  b2[:, g * _H:(g + 1) * _H]]
    wxz, whz, bz, wxr, whr, br, wxn, whn, bn = parts
    sc = lambda i: jnp.minimum(i, _NGRID - 1)
    pj = lambda i: jnp.maximum(i - _NGRID, 0)
    return pl.pallas_call(
        _fused_body,
        grid=(_NGRID + _NVB,),
        in_specs=[
            pl.BlockSpec((_CHUNK, _B, 2 * _D), lambda i: (sc(i), 0, 0)),
            pl.BlockSpec((1, _B, _CHUNK), lambda i: (sc(i), 0, 0)),
            _wspec(), _wspec(), _wspec(),
            _wspec(), _wspec(), _wspec(),
            _bspec(), _bspec(), _bspec(),
            _wspec(), _bspec(),
            pl.BlockSpec(memory_space=pltpu.SMEM),
            pl.BlockSpec((_H, _VB), lambda i: (0, pj(i))),
            pl.BlockSpec((1, _VB), lambda i: (0, pj(i))),
        ],
        out_specs=pl.BlockSpec((_T, _VB, _B), lambda i: (0, pj(i), 0)),
        out_shape=jax.ShapeDtypeStruct((_T, _V, _B), jnp.float32),
        scratch_shapes=[pltpu.VMEM((_B, _H), jnp.float32),
                        pltpu.VMEM((_T * _B, _H), jnp.float32)],
    )(xe2, par, wxz, wxr, wxn, whz, whr, whn, bz, br, bn, Wf, bf, dts,
      Wout, bout)


def kernel(x, t_span, emb_table, Wx, Wh, b, Wf, bf, Wout, bout):
    xi = x.astype(jnp.int32)
    xt = xi.T.reshape(-1)
    idxp = jnp.where(xt < _V // 2, xt, xt - _V // 2)
    par = jnp.transpose(
        (xi >= _V // 2).astype(jnp.float32).reshape(_B, _NGRID, _CHUNK),
        (1, 0, 2))
    table2 = _repack(emb_table)
    xe2 = _sc_gather(idxp, table2)
    dts = t_span[1:] - t_span[:-1]
    out = _fused_call(xe2.reshape(_S, _B, 2 * _D), par, Wx, Wh, b,
                      Wf, bf.reshape(1, -1), dts, Wout,
                      bout.reshape(1, -1))
    return jnp.transpose(out, (0, 2, 1))


# speedup vs baseline: 2.3175x; 1.0110x over previous
import functools

import jax
import jax.numpy as jnp
from jax import lax
from jax.experimental import pallas as pl
from jax.experimental.pallas import tpu as pltpu
from jax.experimental.pallas import tpu_sc as plsc

_B, _S, _V, _D, _H, _T = 128, 200, 100000, 64, 64, 4
_ROWS = _B * _S
_NW = 32
_RPW = _ROWS // _NW
_NCH = 8
_CH = _RPW // _NCH

_CHUNK = 100
_NGRID = _S // _CHUNK
_NSPL = 4
_VB = 2048
_NVB = (_V + _VB - 1) // _VB

def _dot(a, b):
    return jnp.dot(a, b, preferred_element_type=jnp.float32)


def _sigmoid(u):
    return 0.5 * jnp.tanh(0.5 * u) + 0.5


_RB = 2000
_NRB = _V // (2 * _RB)


def _repack_body(lo_ref, hi_ref, t2_ref):
    t2_ref[:, :_D] = lo_ref[...]
    t2_ref[:, _D:] = hi_ref[...]


@jax.jit
def _repack(emb):
    return pl.pallas_call(
        _repack_body,
        grid=(_NRB,),
        in_specs=[
            pl.BlockSpec((_RB, _D), lambda i: (i, 0)),
            pl.BlockSpec((_RB, _D), lambda i: (i + _NRB, 0)),
        ],
        out_specs=pl.BlockSpec((_RB, 2 * _D), lambda i: (i, 0)),
        out_shape=jax.ShapeDtypeStruct((_V // 2, 2 * _D), jnp.float32),
    )(emb, emb)


def _gather_body(idx_hbm, table_hbm, out_hbm, idx_v, rows_v, sem):
    nc = plsc.get_sparse_core_info().num_cores
    wid = lax.axis_index("s") * nc + lax.axis_index("c")
    pltpu.sync_copy(idx_hbm.at[wid], idx_v)
    copies = [
        pltpu.async_copy(table_hbm.at[idx_v.at[j]],
                         rows_v.at[pl.ds(j * _CH, _CH)], sem)
        for j in range(_NCH)
    ]
    for c in copies:
        c.wait()
    pltpu.sync_copy(rows_v, out_hbm.at[pl.ds(wid * _RPW, _RPW)])


@jax.jit
def _sc_gather(idx, table2):
    k = pl.kernel(
        _gather_body,
        mesh=plsc.VectorSubcoreMesh(core_axis_name="c", subcore_axis_name="s"),
        out_type=jax.ShapeDtypeStruct((_ROWS, 2 * _D), jnp.float32),
        scratch_types=[
            pltpu.VMEM((_NCH, _CH), jnp.int32),
            pltpu.VMEM((_RPW, 2 * _D), jnp.float32),
            pltpu.SemaphoreType.DMA,
        ],
    )
    return k(idx.reshape(_NW, _NCH, _CH), table2)


def _fused_body(xe_ref, par_ref, wxz_ref, wxr_ref, wxn_ref,
                whz_ref, whr_ref, whn_ref, bz_ref, br_ref, bn_ref,
                wf_ref, bf_ref, dts_ref, wout_ref, bout_ref,
                out_ref, h_ref, hs_ref):
    i = pl.program_id(0)

    @pl.when(i == 0)
    def _():
        h_ref[...] = jnp.zeros_like(h_ref)

    @pl.when(i < _NGRID)
    def _scan_phase():
        _scan_chunk(xe_ref, par_ref, wxz_ref, wxr_ref, wxn_ref,
                    whz_ref, whr_ref, whn_ref, bz_ref, br_ref, bn_ref,
                    wf_ref, bf_ref, dts_ref, hs_ref, h_ref, i)

    @pl.when(i >= _NGRID)
    def _proj_phase():
        w = wout_ref[...]
        ones = jnp.ones((_B, 1), jnp.float32)
        bias = jax.lax.dot_general(
            bout_ref[...], ones, (((0,), (1,)), ((), ())),
            preferred_element_type=jnp.float32)
        for t in range(_T):
            hs_t = hs_ref[t * _B:(t + 1) * _B]
            out_ref[t] = jax.lax.dot_general(
                w, hs_t, (((0,), (1,)), ((), ())),
                preferred_element_type=jnp.float32) + bias


def _scan_chunk(xe_ref, par_ref, wxz_ref, wxr_ref, wxn_ref,
                whz_ref, whr_ref, whn_ref, bz_ref, br_ref, bn_ref,
                wf_ref, bf_ref, dts_ref, hs_ref, h_ref, i):

    par = par_ref[0]
    sels = []
    for t in range(_CHUNK):
        xt = xe_ref[t]
        p = par[:, t:t + 1]
        lo = xt[:, :_D]
        hi = xt[:, _D:]
        sels.append(lo + p * (hi - lo))
    xe_c = jnp.concatenate(sels, axis=0)
    gxz = _dot(xe_c, wxz_ref[...]) + bz_ref[...]
    gxr = _dot(xe_c, wxr_ref[...]) + br_ref[...]
    gxn = _dot(xe_c, wxn_ref[...]) + bn_ref[...]

    whz, whr, whn = whz_ref[...], whr_ref[...], whn_ref[...]

    def dots(hh):
        return _dot(hh, whz), _dot(hh, whr), _dot(hh, whn)

    sb = _B // _NSPL

    def gates(t, base, gh, hh):
        lo_ = t * _B + base
        hi_ = lo_ + sb
        z = _sigmoid(gxz[lo_:hi_] + gh[0])
        r = _sigmoid(gxr[lo_:hi_] + gh[1])
        n = jnp.tanh(gxn[lo_:hi_] + r * gh[2])
        return hh + z * (n - hh)

    hs_c = [h_ref[k * sb:(k + 1) * sb, :] for k in range(_NSPL)]
    gh_c = [dots(hc) for hc in hs_c]
    for t in range(_CHUNK):
        for k in range(_NSPL):
            hs_c[k] = gates(t, k * sb, gh_c[k], hs_c[k])
            if t + 1 < _CHUNK:
                gh_c[k] = dots(hs_c[k])
    for k in range(_NSPL):
        h_ref[k * sb:(k + 1) * sb, :] = hs_c[k]

    @pl.when(i == _NGRID - 1)
    def _():
        def f(hh):
            return jnp.tanh(_dot(hh, wf_ref[...]) + bf_ref[...])

        h = jnp.concatenate(hs_c, axis=0)
        hs_ref[0:_B, :] = h
        hc = h
        for s in range(_T - 1):
            dt = dts_ref[s]
            k1 = f(hc)
            k2 = f(hc + 0.5 * dt * k1)
            k3 = f(hc + 0.5 * dt * k2)
            k4 = f(hc + dt * k3)
            hc = hc + (dt / 6.0) * (k1 + 2.0 * k2 + 2.0 * k3 + k4)
            hs_ref[(s + 1) * _B:(s + 2) * _B, :] = hc


def _wspec():
    return pl.BlockSpec((_H, _H), lambda i: (0, 0))


def _bspec():
    return pl.BlockSpec((1, _H), lambda i: (0, 0))


@jax.jit
def _fused_call(xe2, par, Wx, Wh, b, Wf, bf, dts, Wout, bout):
    b2 = b.reshape(1, 3 * _H)
    parts = []
    for g in range(3):
        parts += [Wx[:, g * _H:(g + 1) * _H], Wh[:, g * _H:(g + 1) * _H],
                  b2[:, g * _H:(g + 1) * _H]]
    wxz, whz, bz, wxr, whr, br, wxn, whn, bn = parts
    sc = lambda i: jnp.minimum(i, _NGRID - 1)
    pj = lambda i: jnp.maximum(i - _NGRID, 0)
    return pl.pallas_call(
        _fused_body,
        grid=(_NGRID + _NVB,),
        in_specs=[
            pl.BlockSpec((_CHUNK, _B, 2 * _D), lambda i: (sc(i), 0, 0)),
            pl.BlockSpec((1, _B, _CHUNK), lambda i: (sc(i), 0, 0)),
            _wspec(), _wspec(), _wspec(),
            _wspec(), _wspec(), _wspec(),
            _bspec(), _bspec(), _bspec(),
            _wspec(), _bspec(),
            pl.BlockSpec(memory_space=pltpu.SMEM),
            pl.BlockSpec((_H, _VB), lambda i: (0, pj(i))),
            pl.BlockSpec((1, _VB), lambda i: (0, pj(i))),
        ],
        out_specs=pl.BlockSpec((_T, _VB, _B), lambda i: (0, pj(i), 0)),
        out_shape=jax.ShapeDtypeStruct((_T, _V, _B), jnp.float32),
        scratch_shapes=[pltpu.VMEM((_B, _H), jnp.float32),
                        pltpu.VMEM((_T * _B, _H), jnp.float32)],
    )(xe2, par, wxz, wxr, wxn, whz, whr, whn, bz, br, bn, Wf, bf, dts,
      Wout, bout)


def kernel(x, t_span, emb_table, Wx, Wh, b, Wf, bf, Wout, bout):
    xi = x.astype(jnp.int32)
    xt = xi.T.reshape(-1)
    idxp = jnp.where(xt < _V // 2, xt, xt - _V // 2)
    par = jnp.transpose(
        (xi >= _V // 2).astype(jnp.float32).reshape(_B, _NGRID, _CHUNK),
        (1, 0, 2))
    table2 = _repack(emb_table)
    xe2 = _sc_gather(idxp, table2)
    dts = t_span[1:] - t_span[:-1]
    out = _fused_call(xe2.reshape(_S, _B, 2 * _D), par, Wx, Wh, b,
                      Wf, bf.reshape(1, -1), dts, Wout,
                      bout.reshape(1, -1))
    return jnp.transpose(out, (0, 2, 1))


# speedup vs baseline: 2.4719x; 1.0666x over previous
import functools

import jax
import jax.numpy as jnp
from jax import lax
from jax.experimental import pallas as pl
from jax.experimental.pallas import tpu as pltpu
from jax.experimental.pallas import tpu_sc as plsc

_B, _S, _V, _D, _H, _T = 128, 200, 100000, 64, 64, 4
_ROWS = _B * _S
_NW = 32
_RPW = _ROWS // _NW
_NCH = 8
_CH = _RPW // _NCH

_CHUNK = 100
_NGRID = _S // _CHUNK
_NSPL = 4
_VB = 4096
_NVB = (_V + _VB - 1) // _VB

def _dot(a, b):
    return jnp.dot(a, b, preferred_element_type=jnp.float32)


def _sigmoid(u):
    return 0.5 * jnp.tanh(0.5 * u) + 0.5


_RB = 2000
_NRB = _V // (2 * _RB)


def _repack_body(lo_ref, hi_ref, t2_ref):
    t2_ref[:, :_D] = lo_ref[...]
    t2_ref[:, _D:] = hi_ref[...]


@jax.jit
def _repack(emb):
    return pl.pallas_call(
        _repack_body,
        grid=(_NRB,),
        in_specs=[
            pl.BlockSpec((_RB, _D), lambda i: (i, 0)),
            pl.BlockSpec((_RB, _D), lambda i: (i + _NRB, 0)),
        ],
        out_specs=pl.BlockSpec((_RB, 2 * _D), lambda i: (i, 0)),
        out_shape=jax.ShapeDtypeStruct((_V // 2, 2 * _D), jnp.float32),
    )(emb, emb)


def _gather_body(idx_hbm, table_hbm, out_hbm, idx_v, rows_v, sem):
    nc = plsc.get_sparse_core_info().num_cores
    wid = lax.axis_index("s") * nc + lax.axis_index("c")
    pltpu.sync_copy(idx_hbm.at[wid], idx_v)
    copies = [
        pltpu.async_copy(table_hbm.at[idx_v.at[j]],
                         rows_v.at[pl.ds(j * _CH, _CH)], sem)
        for j in range(_NCH)
    ]
    for c in copies:
        c.wait()
    pltpu.sync_copy(rows_v, out_hbm.at[pl.ds(wid * _RPW, _RPW)])


@jax.jit
def _sc_gather(idx, table2):
    k = pl.kernel(
        _gather_body,
        mesh=plsc.VectorSubcoreMesh(core_axis_name="c", subcore_axis_name="s"),
        out_type=jax.ShapeDtypeStruct((_ROWS, 2 * _D), jnp.float32),
        scratch_types=[
            pltpu.VMEM((_NCH, _CH), jnp.int32),
            pltpu.VMEM((_RPW, 2 * _D), jnp.float32),
            pltpu.SemaphoreType.DMA,
        ],
    )
    return k(idx.reshape(_NW, _NCH, _CH), table2)


def _fused_body(xe_ref, par_ref, wxz_ref, wxr_ref, wxn_ref,
                whz_ref, whr_ref, whn_ref, bz_ref, br_ref, bn_ref,
                wf_ref, bf_ref, dts_ref, wout_ref, bout_ref,
                out_ref, h_ref, hs_ref):
    i = pl.program_id(0)

    @pl.when(i == 0)
    def _():
        h_ref[...] = jnp.zeros_like(h_ref)

    @pl.when(i < _NGRID)
    def _scan_phase():
        _scan_chunk(xe_ref, par_ref, wxz_ref, wxr_ref, wxn_ref,
                    whz_ref, whr_ref, whn_ref, bz_ref, br_ref, bn_ref,
                    wf_ref, bf_ref, dts_ref, hs_ref, h_ref, i)

    @pl.when(i >= _NGRID)
    def _proj_phase():
        w = wout_ref[...]
        ones = jnp.ones((_B, 1), jnp.float32)
        bias = jax.lax.dot_general(
            bout_ref[...], ones, (((0,), (1,)), ((), ())),
            preferred_element_type=jnp.float32)
        for t in range(_T):
            hs_t = hs_ref[t * _B:(t + 1) * _B]
            out_ref[t] = jax.lax.dot_general(
                w, hs_t, (((0,), (1,)), ((), ())),
                preferred_element_type=jnp.float32) + bias


def _scan_chunk(xe_ref, par_ref, wxz_ref, wxr_ref, wxn_ref,
                whz_ref, whr_ref, whn_ref, bz_ref, br_ref, bn_ref,
                wf_ref, bf_ref, dts_ref, hs_ref, h_ref, i):

    par = par_ref[0]
    sels = []
    for t in range(_CHUNK):
        xt = xe_ref[t]
        p = par[:, t:t + 1]
        lo = xt[:, :_D]
        hi = xt[:, _D:]
        sels.append(lo + p * (hi - lo))
    xe_c = jnp.concatenate(sels, axis=0)
    gxz = _dot(xe_c, wxz_ref[...]) + bz_ref[...]
    gxr = _dot(xe_c, wxr_ref[...]) + br_ref[...]
    gxn = _dot(xe_c, wxn_ref[...]) + bn_ref[...]

    whz, whr, whn = whz_ref[...], whr_ref[...], whn_ref[...]

    def dots(hh):
        return _dot(hh, whz), _dot(hh, whr), _dot(hh, whn)

    sb = _B // _NSPL

    def gates(t, base, gh, hh):
        lo_ = t * _B + base
        hi_ = lo_ + sb
        z = _sigmoid(gxz[lo_:hi_] + gh[0])
        r = _sigmoid(gxr[lo_:hi_] + gh[1])
        n = jnp.tanh(gxn[lo_:hi_] + r * gh[2])
        return hh + z * (n - hh)

    hs_c = [h_ref[k * sb:(k + 1) * sb, :] for k in range(_NSPL)]
    gh_c = [dots(hc) for hc in hs_c]
    for t in range(_CHUNK):
        for k in range(_NSPL):
            hs_c[k] = gates(t, k * sb, gh_c[k], hs_c[k])
            if t + 1 < _CHUNK:
                gh_c[k] = dots(hs_c[k])
    for k in range(_NSPL):
        h_ref[k * sb:(k + 1) * sb, :] = hs_c[k]

    @pl.when(i == _NGRID - 1)
    def _():
        def f(hh):
            return jnp.tanh(_dot(hh, wf_ref[...]) + bf_ref[...])

        h = jnp.concatenate(hs_c, axis=0)
        hs_ref[0:_B, :] = h
        hc = h
        for s in range(_T - 1):
            dt = dts_ref[s]
            k1 = f(hc)
            k2 = f(hc + 0.5 * dt * k1)
            k3 = f(hc + 0.5 * dt * k2)
            k4 = f(hc + dt * k3)
            hc = hc + (dt / 6.0) * (k1 + 2.0 * k2 + 2.0 * k3 + k4)
            hs_ref[(s + 1) * _B:(s + 2) * _B, :] = hc


def _wspec():
    return pl.BlockSpec((_H, _H), lambda i: (0, 0))


def _bspec():
    return pl.BlockSpec((1, _H), lambda i: (0, 0))


@jax.jit
def _fused_call(xe2, par, Wx, Wh, b, Wf, bf, dts, Wout, bout):
    b2 = b.reshape(1, 3 * _H)
    parts = []
    for g in range(3):
        parts += [Wx[:, g * _H:(g + 1) * _H], Wh[:, g * _H:(g + 1) * _H],
                  b2[:, g * _H:(g + 1) * _H]]
    wxz, whz, bz, wxr, whr, br, wxn, whn, bn = parts
    sc = lambda i: jnp.minimum(i, _NGRID - 1)
    pj = lambda i: jnp.maximum(i - _NGRID, 0)
    return pl.pallas_call(
        _fused_body,
        grid=(_NGRID + _NVB,),
        in_specs=[
            pl.BlockSpec((_CHUNK, _B, 2 * _D), lambda i: (sc(i), 0, 0)),
            pl.BlockSpec((1, _B, _CHUNK), lambda i: (sc(i), 0, 0)),
            _wspec(), _wspec(), _wspec(),
            _wspec(), _wspec(), _wspec(),
            _bspec(), _bspec(), _bspec(),
            _wspec(), _bspec(),
            pl.BlockSpec(memory_space=pltpu.SMEM),
            pl.BlockSpec((_H, _VB), lambda i: (0, pj(i))),
            pl.BlockSpec((1, _VB), lambda i: (0, pj(i))),
        ],
        out_specs=pl.BlockSpec((_T, _VB, _B), lambda i: (0, pj(i), 0)),
        out_shape=jax.ShapeDtypeStruct((_T, _V, _B), jnp.float32),
        scratch_shapes=[pltpu.VMEM((_B, _H), jnp.float32),
                        pltpu.VMEM((_T * _B, _H), jnp.float32)],
    )(xe2, par, wxz, wxr, wxn, whz, whr, whn, bz, br, bn, Wf, bf, dts,
      Wout, bout)


def kernel(x, t_span, emb_table, Wx, Wh, b, Wf, bf, Wout, bout):
    xi = x.astype(jnp.int32)
    xt = xi.T.reshape(-1)
    idxp = jnp.where(xt < _V // 2, xt, xt - _V // 2)
    par = jnp.transpose(
        (xi >= _V // 2).astype(jnp.float32).reshape(_B, _NGRID, _CHUNK),
        (1, 0, 2))
    table2 = _repack(emb_table)
    xe2 = _sc_gather(idxp, table2)
    dts = t_span[1:] - t_span[:-1]
    out = _fused_call(xe2.reshape(_S, _B, 2 * _D), par, Wx, Wh, b,
                      Wf, bf.reshape(1, -1), dts, Wout,
                      bout.reshape(1, -1))
    return jnp.transpose(out, (0, 2, 1))


# speedup vs baseline: 2.4794x; 1.0030x over previous
import functools

import jax
import jax.numpy as jnp
from jax import lax
from jax.experimental import pallas as pl
from jax.experimental.pallas import tpu as pltpu
from jax.experimental.pallas import tpu_sc as plsc

_B, _S, _V, _D, _H, _T = 128, 200, 100000, 64, 64, 4
_ROWS = _B * _S
_NW = 32
_RPW = _ROWS // _NW
_NCH = 8
_CH = _RPW // _NCH

_CHUNK = 100
_NGRID = _S // _CHUNK
_NSPL = 4
_VB = 6144
_NVB = (_V + _VB - 1) // _VB

def _dot(a, b):
    return jnp.dot(a, b, preferred_element_type=jnp.float32)


def _sigmoid(u):
    return 0.5 * jnp.tanh(0.5 * u) + 0.5


_RB = 2000
_NRB = _V // (2 * _RB)


def _repack_body(lo_ref, hi_ref, t2_ref):
    t2_ref[:, :_D] = lo_ref[...]
    t2_ref[:, _D:] = hi_ref[...]


@jax.jit
def _repack(emb):
    return pl.pallas_call(
        _repack_body,
        grid=(_NRB,),
        in_specs=[
            pl.BlockSpec((_RB, _D), lambda i: (i, 0)),
            pl.BlockSpec((_RB, _D), lambda i: (i + _NRB, 0)),
        ],
        out_specs=pl.BlockSpec((_RB, 2 * _D), lambda i: (i, 0)),
        out_shape=jax.ShapeDtypeStruct((_V // 2, 2 * _D), jnp.float32),
    )(emb, emb)


def _gather_body(idx_hbm, table_hbm, out_hbm, idx_v, rows_v, sem):
    nc = plsc.get_sparse_core_info().num_cores
    wid = lax.axis_index("s") * nc + lax.axis_index("c")
    pltpu.sync_copy(idx_hbm.at[wid], idx_v)
    copies = [
        pltpu.async_copy(table_hbm.at[idx_v.at[j]],
                         rows_v.at[pl.ds(j * _CH, _CH)], sem)
        for j in range(_NCH)
    ]
    for c in copies:
        c.wait()
    pltpu.sync_copy(rows_v, out_hbm.at[pl.ds(wid * _RPW, _RPW)])


@jax.jit
def _sc_gather(idx, table2):
    k = pl.kernel(
        _gather_body,
        mesh=plsc.VectorSubcoreMesh(core_axis_name="c", subcore_axis_name="s"),
        out_type=jax.ShapeDtypeStruct((_ROWS, 2 * _D), jnp.float32),
        scratch_types=[
            pltpu.VMEM((_NCH, _CH), jnp.int32),
            pltpu.VMEM((_RPW, 2 * _D), jnp.float32),
            pltpu.SemaphoreType.DMA,
        ],
    )
    return k(idx.reshape(_NW, _NCH, _CH), table2)


def _fused_body(xe_ref, par_ref, wxz_ref, wxr_ref, wxn_ref,
                whz_ref, whr_ref, whn_ref, bz_ref, br_ref, bn_ref,
                wf_ref, bf_ref, dts_ref, wout_ref, bout_ref,
                out_ref, h_ref, hs_ref):
    i = pl.program_id(0)

    @pl.when(i == 0)
    def _():
        h_ref[...] = jnp.zeros_like(h_ref)

    @pl.when(i < _NGRID)
    def _scan_phase():
        _scan_chunk(xe_ref, par_ref, wxz_ref, wxr_ref, wxn_ref,
                    whz_ref, whr_ref, whn_ref, bz_ref, br_ref, bn_ref,
                    wf_ref, bf_ref, dts_ref, hs_ref, h_ref, i)

    @pl.when(i >= _NGRID)
    def _proj_phase():
        w = wout_ref[...]
        ones = jnp.ones((_B, 1), jnp.float32)
        bias = jax.lax.dot_general(
            bout_ref[...], ones, (((0,), (1,)), ((), ())),
            preferred_element_type=jnp.float32)
        for t in range(_T):
            hs_t = hs_ref[t * _B:(t + 1) * _B]
            out_ref[t] = jax.lax.dot_general(
                w, hs_t, (((0,), (1,)), ((), ())),
                preferred_element_type=jnp.float32) + bias


def _scan_chunk(xe_ref, par_ref, wxz_ref, wxr_ref, wxn_ref,
                whz_ref, whr_ref, whn_ref, bz_ref, br_ref, bn_ref,
                wf_ref, bf_ref, dts_ref, hs_ref, h_ref, i):

    par = par_ref[0]
    sels = []
    for t in range(_CHUNK):
        xt = xe_ref[t]
        p = par[:, t:t + 1]
        lo = xt[:, :_D]
        hi = xt[:, _D:]
        sels.append(lo + p * (hi - lo))
    xe_c = jnp.concatenate(sels, axis=0)
    gxz = _dot(xe_c, wxz_ref[...]) + bz_ref[...]
    gxr = _dot(xe_c, wxr_ref[...]) + br_ref[...]
    gxn = _dot(xe_c, wxn_ref[...]) + bn_ref[...]

    whz, whr, whn = whz_ref[...], whr_ref[...], whn_ref[...]

    def dots(hh):
        return _dot(hh, whz), _dot(hh, whr), _dot(hh, whn)

    sb = _B // _NSPL

    def gates(t, base, gh, hh):
        lo_ = t * _B + base
        hi_ = lo_ + sb
        z = _sigmoid(gxz[lo_:hi_] + gh[0])
        r = _sigmoid(gxr[lo_:hi_] + gh[1])
        n = jnp.tanh(gxn[lo_:hi_] + r * gh[2])
        return hh + z * (n - hh)

    hs_c = [h_ref[k * sb:(k + 1) * sb, :] for k in range(_NSPL)]
    gh_c = [dots(hc) for hc in hs_c]
    for t in range(_CHUNK):
        for k in range(_NSPL):
            hs_c[k] = gates(t, k * sb, gh_c[k], hs_c[k])
            if t + 1 < _CHUNK:
                gh_c[k] = dots(hs_c[k])
    for k in range(_NSPL):
        h_ref[k * sb:(k + 1) * sb, :] = hs_c[k]

    @pl.when(i == _NGRID - 1)
    def _():
        def f(hh):
            return jnp.tanh(_dot(hh, wf_ref[...]) + bf_ref[...])

        h = jnp.concatenate(hs_c, axis=0)
        hs_ref[0:_B, :] = h
        hc = h
        for s in range(_T - 1):
            dt = dts_ref[s]
            k1 = f(hc)
            k2 = f(hc + 0.5 * dt * k1)
            k3 = f(hc + 0.5 * dt * k2)
            k4 = f(hc + dt * k3)
            hc = hc + (dt / 6.0) * (k1 + 2.0 * k2 + 2.0 * k3 + k4)
            hs_ref[(s + 1) * _B:(s + 2) * _B, :] = hc


def _wspec():
    return pl.BlockSpec((_H, _H), lambda i: (0, 0))


def _bspec():
    return pl.BlockSpec((1, _H), lambda i: (0, 0))


@jax.jit
def _fused_call(xe2, par, Wx, Wh, b, Wf, bf, dts, Wout, bout):
    b2 = b.reshape(1, 3 * _H)
    parts = []
    for g in range(3):
        parts += [Wx[:, g * _H:(g + 1) * _H], Wh[:, g * _H:(g + 1) * _H],
                  b2[:, g * _H:(g + 1) * _H]]
    wxz, whz, bz, wxr, whr, br, wxn, whn, bn = parts
    sc = lambda i: jnp.minimum(i, _NGRID - 1)
    pj = lambda i: jnp.maximum(i - _NGRID, 0)
    return pl.pallas_call(
        _fused_body,
        grid=(_NGRID + _NVB,),
        in_specs=[
            pl.BlockSpec((_CHUNK, _B, 2 * _D), lambda i: (sc(i), 0, 0)),
            pl.BlockSpec((1, _B, _CHUNK), lambda i: (sc(i), 0, 0)),
            _wspec(), _wspec(), _wspec(),
            _wspec(), _wspec(), _wspec(),
            _bspec(), _bspec(), _bspec(),
            _wspec(), _bspec(),
            pl.BlockSpec(memory_space=pltpu.SMEM),
            pl.BlockSpec((_H, _VB), lambda i: (0, pj(i))),
            pl.BlockSpec((1, _VB), lambda i: (0, pj(i))),
        ],
        out_specs=pl.BlockSpec((_T, _VB, _B), lambda i: (0, pj(i), 0)),
        out_shape=jax.ShapeDtypeStruct((_T, _V, _B), jnp.float32),
        scratch_shapes=[pltpu.VMEM((_B, _H), jnp.float32),
                        pltpu.VMEM((_T * _B, _H), jnp.float32)],
    )(xe2, par, wxz, wxr, wxn, whz, whr, whn, bz, br, bn, Wf, bf, dts,
      Wout, bout)


def kernel(x, t_span, emb_table, Wx, Wh, b, Wf, bf, Wout, bout):
    xi = x.astype(jnp.int32)
    xt = xi.T.reshape(-1)
    idxp = jnp.where(xt < _V // 2, xt, xt - _V // 2)
    par = jnp.transpose(
        (xi >= _V // 2).astype(jnp.float32).reshape(_B, _NGRID, _CHUNK),
        (1, 0, 2))
    table2 = _repack(emb_table)
    xe2 = _sc_gather(idxp, table2)
    dts = t_span[1:] - t_span[:-1]
    out = _fused_call(xe2.reshape(_S, _B, 2 * _D), par, Wx, Wh, b,
                      Wf, bf.reshape(1, -1), dts, Wout,
                      bout.reshape(1, -1))
    return jnp.transpose(out, (0, 2, 1))


# speedup vs baseline: 2.4865x; 1.0029x over previous
import jax
import jax.numpy as jnp
from jax import lax
from jax.experimental import pallas as pl
from jax.experimental.pallas import tpu as pltpu
from jax.experimental.pallas import tpu_sc as plsc

_B, _S, _V, _D, _H, _T = 128, 200, 100000, 64, 64, 4
_ROWS = _B * _S
_NW = 32
_RPW = _ROWS // _NW
_NCH = 8
_CH = _RPW // _NCH

_CHUNK = 100
_NGRID = _S // _CHUNK
_NSPL = 4
_VB = 6144
_NVB = (_V + _VB - 1) // _VB

def _dot(a, b):
    return jnp.dot(a, b, preferred_element_type=jnp.float32)


def _sigmoid(u):
    return 0.5 * jnp.tanh(0.5 * u) + 0.5


_RB = 2000
_NRB = _V // (2 * _RB)


def _repack_body(lo_ref, hi_ref, t2_ref):
    t2_ref[:, :_D] = lo_ref[...]
    t2_ref[:, _D:] = hi_ref[...]


@jax.jit
def _repack(emb):
    return pl.pallas_call(
        _repack_body,
        grid=(_NRB,),
        in_specs=[
            pl.BlockSpec((_RB, _D), lambda i: (i, 0)),
            pl.BlockSpec((_RB, _D), lambda i: (i + _NRB, 0)),
        ],
        out_specs=pl.BlockSpec((_RB, 2 * _D), lambda i: (i, 0)),
        out_shape=jax.ShapeDtypeStruct((_V // 2, 2 * _D), jnp.float32),
    )(emb, emb)


def _gather_body(idx_hbm, table_hbm, out_hbm, idx_v, rows_v, sem):
    nc = plsc.get_sparse_core_info().num_cores
    wid = lax.axis_index("s") * nc + lax.axis_index("c")
    pltpu.sync_copy(idx_hbm.at[wid], idx_v)
    copies = [
        pltpu.async_copy(table_hbm.at[idx_v.at[j]],
                         rows_v.at[pl.ds(j * _CH, _CH)], sem)
        for j in range(_NCH)
    ]
    for c in copies:
        c.wait()
    pltpu.sync_copy(rows_v, out_hbm.at[pl.ds(wid * _RPW, _RPW)])


@jax.jit
def _sc_gather(idx, table2):
    k = pl.kernel(
        _gather_body,
        mesh=plsc.VectorSubcoreMesh(core_axis_name="c", subcore_axis_name="s"),
        out_type=jax.ShapeDtypeStruct((_ROWS, 2 * _D), jnp.float32),
        scratch_types=[
            pltpu.VMEM((_NCH, _CH), jnp.int32),
            pltpu.VMEM((_RPW, 2 * _D), jnp.float32),
            pltpu.SemaphoreType.DMA,
        ],
    )
    return k(idx.reshape(_NW, _NCH, _CH), table2)


def _fused_body(xe_ref, par_ref, wxz_ref, wxr_ref, wxn_ref,
                whz_ref, whr_ref, whn_ref, bz_ref, br_ref, bn_ref,
                wf_ref, bf_ref, dts_ref, wout_ref, bout_ref,
                out_ref, h_ref, hs_ref):
    i = pl.program_id(0)

    @pl.when(i == 0)
    def _():
        h_ref[...] = jnp.zeros_like(h_ref)

    @pl.when(i < _NGRID)
    def _scan_phase():
        _scan_chunk(xe_ref, par_ref, wxz_ref, wxr_ref, wxn_ref,
                    whz_ref, whr_ref, whn_ref, bz_ref, br_ref, bn_ref,
                    wf_ref, bf_ref, dts_ref, hs_ref, h_ref, i)

    @pl.when(i >= _NGRID)
    def _proj_phase():
        w = wout_ref[...]
        ones = jnp.ones((_B, 1), jnp.float32)
        bias = jax.lax.dot_general(
            bout_ref[...], ones, (((0,), (1,)), ((), ())),
            preferred_element_type=jnp.float32)
        for t in range(_T):
            hs_t = hs_ref[t * _B:(t + 1) * _B]
            out_ref[t] = jax.lax.dot_general(
                w, hs_t, (((0,), (1,)), ((), ())),
                preferred_element_type=jnp.float32) + bias


def _scan_chunk(xe_ref, par_ref, wxz_ref, wxr_ref, wxn_ref,
                whz_ref, whr_ref, whn_ref, bz_ref, br_ref, bn_ref,
                wf_ref, bf_ref, dts_ref, hs_ref, h_ref, i):

    par = par_ref[0]
    sels = []
    for t in range(_CHUNK):
        xt = xe_ref[t]
        p = par[:, t:t + 1]
        lo = xt[:, :_D]
        hi = xt[:, _D:]
        sels.append(lo + p * (hi - lo))
    xe_c = jnp.concatenate(sels, axis=0)
    gxz = _dot(xe_c, wxz_ref[...]) + bz_ref[...]
    gxr = _dot(xe_c, wxr_ref[...]) + br_ref[...]
    gxn = _dot(xe_c, wxn_ref[...]) + bn_ref[...]

    whz, whr, whn = whz_ref[...], whr_ref[...], whn_ref[...]

    def dots(hh):
        return _dot(hh, whz), _dot(hh, whr), _dot(hh, whn)

    sb = _B // _NSPL

    def gates(t, base, gh, hh):
        lo_ = t * _B + base
        hi_ = lo_ + sb
        z = _sigmoid(gxz[lo_:hi_] + gh[0])
        r = _sigmoid(gxr[lo_:hi_] + gh[1])
        n = jnp.tanh(gxn[lo_:hi_] + r * gh[2])
        return hh + z * (n - hh)

    hs_c = [h_ref[k * sb:(k + 1) * sb, :] for k in range(_NSPL)]
    gh_c = [dots(hc) for hc in hs_c]
    for t in range(_CHUNK):
        for k in range(_NSPL):
            hs_c[k] = gates(t, k * sb, gh_c[k], hs_c[k])
            if t + 1 < _CHUNK:
                gh_c[k] = dots(hs_c[k])
    for k in range(_NSPL):
        h_ref[k * sb:(k + 1) * sb, :] = hs_c[k]

    @pl.when(i == _NGRID - 1)
    def _():
        def f(hh):
            return jnp.tanh(_dot(hh, wf_ref[...]) + bf_ref[...])

        h = jnp.concatenate(hs_c, axis=0)
        hs_ref[0:_B, :] = h
        hc = h
        for s in range(_T - 1):
            dt = dts_ref[s]
            k1 = f(hc)
            k2 = f(hc + 0.5 * dt * k1)
            k3 = f(hc + 0.5 * dt * k2)
            k4 = f(hc + dt * k3)
            hc = hc + (dt / 6.0) * (k1 + 2.0 * k2 + 2.0 * k3 + k4)
            hs_ref[(s + 1) * _B:(s + 2) * _B, :] = hc


def _wspec():
    return pl.BlockSpec((_H, _H), lambda i: (0, 0))


def _bspec():
    return pl.BlockSpec((1, _H), lambda i: (0, 0))


@jax.jit
def _fused_call(xe2, par, Wx, Wh, b, Wf, bf, dts, Wout, bout):
    b2 = b.reshape(1, 3 * _H)
    parts = []
    for g in range(3):
        parts += [Wx[:, g * _H:(g + 1) * _H], Wh[:, g * _H:(g + 1) * _H],
                  b2[:, g * _H:(g + 1) * _H]]
    wxz, whz, bz, wxr, whr, br, wxn, whn, bn = parts
    sc = lambda i: jnp.minimum(i, _NGRID - 1)
    pj = lambda i: jnp.maximum(i - _NGRID, 0)
    return pl.pallas_call(
        _fused_body,
        grid=(_NGRID + _NVB,),
        in_specs=[
            pl.BlockSpec((_CHUNK, _B, 2 * _D), lambda i: (sc(i), 0, 0)),
            pl.BlockSpec((1, _B, _CHUNK), lambda i: (sc(i), 0, 0)),
            _wspec(), _wspec(), _wspec(),
            _wspec(), _wspec(), _wspec(),
            _bspec(), _bspec(), _bspec(),
            _wspec(), _bspec(),
            pl.BlockSpec(memory_space=pltpu.SMEM),
            pl.BlockSpec((_H, _VB), lambda i: (0, pj(i))),
            pl.BlockSpec((1, _VB), lambda i: (0, pj(i))),
        ],
        out_specs=pl.BlockSpec((_T, _VB, _B), lambda i: (0, pj(i), 0)),
        out_shape=jax.ShapeDtypeStruct((_T, _V, _B), jnp.float32),
        scratch_shapes=[pltpu.VMEM((_B, _H), jnp.float32),
                        pltpu.VMEM((_T * _B, _H), jnp.float32)],
    )(xe2, par, wxz, wxr, wxn, whz, whr, whn, bz, br, bn, Wf, bf, dts,
      Wout, bout)


def kernel(x, t_span, emb_table, Wx, Wh, b, Wf, bf, Wout, bout):
    xi = x.astype(jnp.int32)
    xt = xi.T.reshape(-1)
    idxp = jnp.where(xt < _V // 2, xt, xt - _V // 2)
    par = jnp.transpose(
        (xi >= _V // 2).astype(jnp.float32).reshape(_B, _NGRID, _CHUNK),
        (1, 0, 2))
    table2 = _repack(emb_table)
    xe2 = _sc_gather(idxp, table2)
    dts = t_span[1:] - t_span[:-1]
    out = _fused_call(xe2.reshape(_S, _B, 2 * _D), par, Wx, Wh, b,
                      Wf, bf.reshape(1, -1), dts, Wout,
                      bout.reshape(1, -1))
    return jnp.transpose(out, (0, 2, 1))
